# Initial kernel scaffold; baseline (speedup 1.0000x reference)
import jax
import jax.numpy as jnp
from jax.experimental import pallas as pl


def kernel(feature, video_feature, a0_w1, a0_w2, a0_s1, a0_b1, a0_s2, a0_b2, a1_w1, a1_w2, a1_s1, a1_b1, a1_s2, a1_b2, a2_w1, a2_w2, a2_s1, a2_b1, a2_s2, a2_b2, a3_w1, a3_w2, a3_s1, a3_b1, a3_s2, a3_b2, a4_w1, a4_w2, a4_s1, a4_b1, a4_s2, a4_b2, a5_w1, a5_w2, a5_s1, a5_b1, a5_s2, a5_b2, v0_w, v0_b, v1_w, v1_b, v2_w, v2_b, v3_w, v3_b, v4_w, v4_b, v5_w, v5_b, fc_w, fc_b):
    raise NotImplementedError("write your pallas kernel here")


# speedup vs baseline: 6.5603x; 6.5603x over previous
import jax
import jax.numpy as jnp
from jax.experimental import pallas as pl
from jax.experimental.pallas import tpu as pltpu

_UPSAMPLE = 32
_VMEM_LIMIT = 56 * 1024 * 1024


def _band2d(w, W):
    _, _, cin, cout = w.shape
    rel = jnp.arange(W)[:, None] - jnp.arange(W)[None, :] + 1
    ok = (rel >= 0) & (rel <= 2)
    taps = w[:, jnp.clip(rel, 0, 2)] * ok[None, :, :, None, None].astype(w.dtype)
    return taps.transpose(0, 1, 3, 2, 4).reshape(3, W * cin, W * cout)


def _band3d(w, W):
    return jnp.stack([_band2d(w[kt], W) for kt in range(3)])


def _conv_bn_relu(x, w_ref, s_ref, b_ref, Nb, H, K, Nout):
    z = jnp.zeros((Nb, 1, K), jnp.float32)
    xp = jnp.concatenate([z, x, z], axis=1).astype(jnp.bfloat16)
    acc = jnp.zeros((Nb * H, Nout), jnp.float32)
    for kh in range(3):
        acc += jnp.dot(xp[:, kh:kh + H, :].reshape(Nb * H, K), w_ref[kh],
                       preferred_element_type=jnp.float32)
    y = jnp.maximum(acc * s_ref[...] + b_ref[...], 0.0)
    return y.reshape(Nb, H, Nout)


def _avgpool22(x, Nb, H, W, C):
    xr = x.reshape(Nb, H // 2, 2, W * C)
    yr = xr[:, :, 0, :] + xr[:, :, 1, :]
    parts = [yr[..., (2 * w) * C:(2 * w + 1) * C] +
             yr[..., (2 * w + 1) * C:(2 * w + 2) * C] for w in range(W // 2)]
    return 0.25 * jnp.concatenate(parts, axis=-1)


def _conv3d_even_t(x, w_ref, b_ref, Nb, T, H, K, Nout):
    To = T // 2
    zt = jnp.zeros((Nb, 1, H, K), jnp.float32)
    zh = jnp.zeros((Nb, T + 2, 1, K), jnp.float32)
    xp = jnp.concatenate([zt, x, zt], axis=1)
    xp = jnp.concatenate([zh, xp, zh], axis=2).astype(jnp.bfloat16)
    acc = jnp.zeros((Nb * To * H, Nout), jnp.float32)
    for kt in range(3):
        for kh in range(3):
            sl = xp[:, kt:kt + T, kh:kh + H, :].reshape(
                Nb, To, 2, H, K)[:, :, 0].reshape(Nb * To * H, K)
            acc += jnp.dot(sl, w_ref[kt, kh],
                           preferred_element_type=jnp.float32)
    return (acc + b_ref[...]).reshape(Nb, To, H, Nout)


def _maxpool_hw(x, H, W, C, p):
    Hout = (H + 2 * p - 2) // 2 + 1
    Wout = (W + 2 * p - 2) // 2 + 1
    rows = []
    for ho in range(Hout):
        r = None
        for i in (2 * ho - p, 2 * ho - p + 1):
            if 0 <= i < H:
                v = x[:, :, i, :]
                r = v if r is None else jnp.maximum(r, v)
        rows.append(r[:, :, None, :])
    x = jnp.concatenate(rows, axis=2)
    cols = []
    for wo in range(Wout):
        c = None
        for j in (2 * wo - p, 2 * wo - p + 1):
            if 0 <= j < W:
                v = x[..., j * C:(j + 1) * C]
                c = v if c is None else jnp.maximum(c, v)
        cols.append(c)
    return jnp.concatenate(cols, axis=-1), Hout, Wout


def _make_audio_body(Nb, dims):
    def body(*refs):
        x = refs[0][...].astype(jnp.float32)
        o_ref = refs[-1]
        for i, (H, W, Cin, Cout) in enumerate(dims):
            w1, w2, s1, b1, s2, b2 = refs[1 + 6 * i: 7 + 6 * i]
            x = _conv_bn_relu(x, w1, s1, b1, Nb, H, W * Cin, W * Cout)
            x = _conv_bn_relu(x, w2, s2, b2, Nb, H, W * Cout, W * Cout)
            if i < len(dims) - 1:
                x = _avgpool22(x, Nb, H, W, Cout)
        o_ref[...] = x.astype(o_ref.dtype)
    return body


def _make_video_body(Nb, dims, C_last, nclass):
    nL = len(dims)

    def body(*refs):
        x = refs[0][...].astype(jnp.float32)
        a_ref, fw_ref, fb_ref = refs[1 + 2 * nL], refs[2 + 2 * nL], refs[3 + 2 * nL]
        tfv_ref, frame_ref = refs[-2], refs[-1]
        for i, (T, H, W, Cin, Cout, p) in enumerate(dims):
            w_ref, b_ref = refs[1 + 2 * i], refs[2 + 2 * i]
            x = _conv3d_even_t(x, w_ref, b_ref, Nb, T, H, W * Cin, W * Cout)
            x, _, _ = _maxpool_hw(x, H, W, Cout, p)
        Tf, Hf = x.shape[1], x.shape[2]
        Wf = x.shape[3] // C_last
        acc = x[:, :, 0, :]
        for h in range(1, Hf):
            acc = acc + x[:, :, h, :]
        m = acc[..., :C_last]
        for w in range(1, Wf):
            m = m + acc[..., w * C_last:(w + 1) * C_last]
        tfv = m * (1.0 / float(Hf * Wf))
        tfv_ref[...] = tfv.astype(tfv_ref.dtype)
        a = a_ref[...].astype(jnp.float32)
        fused = jnp.maximum(jnp.max(a, axis=1), jnp.max(tfv, axis=1))
        z = jnp.dot(fused, fw_ref[...],
                    preferred_element_type=jnp.float32) + fb_ref[...]
        frame = 1.0 / (1.0 + jnp.exp(-z))
        frame_ref[...] = frame[:, None, :].astype(frame_ref.dtype)
    return body


def kernel(feature, video_feature, a0_w1, a0_w2, a0_s1, a0_b1, a0_s2, a0_b2,
           a1_w1, a1_w2, a1_s1, a1_b1, a1_s2, a1_b2,
           a2_w1, a2_w2, a2_s1, a2_b1, a2_s2, a2_b2,
           a3_w1, a3_w2, a3_s1, a3_b1, a3_s2, a3_b2,
           a4_w1, a4_w2, a4_s1, a4_b1, a4_s2, a4_b2,
           a5_w1, a5_w2, a5_s1, a5_b1, a5_s2, a5_b2,
           v0_w, v0_b, v1_w, v1_b, v2_w, v2_b,
           v3_w, v3_b, v4_w, v4_b, v5_w, v5_b, fc_w, fc_b):
    ablocks = [(a0_w1, a0_w2, a0_s1, a0_b1, a0_s2, a0_b2),
               (a1_w1, a1_w2, a1_s1, a1_b1, a1_s2, a1_b2),
               (a2_w1, a2_w2, a2_s1, a2_b1, a2_s2, a2_b2),
               (a3_w1, a3_w2, a3_s1, a3_b1, a3_s2, a3_b2),
               (a4_w1, a4_w2, a4_s1, a4_b1, a4_s2, a4_b2),
               (a5_w1, a5_w2, a5_s1, a5_b1, a5_s2, a5_b2)]
    vconvs = [(v0_w, v0_b), (v1_w, v1_b), (v2_w, v2_b),
              (v3_w, v3_b), (v4_w, v4_b), (v5_w, v5_b)]

    N, Ta, Fa = feature.shape
    _, Cv, Tv, Hv, Wv = video_feature.shape
    nclass = fc_w.shape[0]

    Nba = 8 if N % 8 == 0 else 1
    adims, ops, specs = [], [feature], [
        pl.BlockSpec((Nba, Ta, Fa), lambda n: (n, 0, 0))]
    H, W, Cin = Ta, Fa, 1
    for i, (w1, w2, s1, b1, s2, b2) in enumerate(ablocks):
        Cout = w1.shape[3]
        adims.append((H, W, Cin, Cout))
        ops += [_band2d(w1, W).astype(jnp.bfloat16),
                _band2d(w2, W).astype(jnp.bfloat16),
                jnp.tile(s1, W)[None, :], jnp.tile(b1, W)[None, :],
                jnp.tile(s2, W)[None, :], jnp.tile(b2, W)[None, :]]
        specs += [pl.BlockSpec(ops[-6].shape, lambda n: (0, 0, 0)),
                  pl.BlockSpec(ops[-5].shape, lambda n: (0, 0, 0)),
                  pl.BlockSpec(ops[-4].shape, lambda n: (0, 0)),
                  pl.BlockSpec(ops[-3].shape, lambda n: (0, 0)),
                  pl.BlockSpec(ops[-2].shape, lambda n: (0, 0)),
                  pl.BlockSpec(ops[-1].shape, lambda n: (0, 0))]
        if i < len(ablocks) - 1:
            H, W = H // 2, W // 2
        Cin = Cout
    C = Cin
    tf_a = pl.pallas_call(
        _make_audio_body(Nba, adims),
        out_shape=jax.ShapeDtypeStruct((N, H, C), feature.dtype),
        grid_spec=pltpu.PrefetchScalarGridSpec(
            num_scalar_prefetch=0, grid=(N // Nba,), in_specs=specs,
            out_specs=pl.BlockSpec((Nba, H, C), lambda n: (n, 0, 0))),
        compiler_params=pltpu.CompilerParams(
            dimension_semantics=("parallel",),
            vmem_limit_bytes=_VMEM_LIMIT),
    )(*ops)

    Nbv = 4 if N % 4 == 0 else 1
    xv = jnp.transpose(video_feature, (0, 2, 3, 4, 1)).reshape(N, Tv, Hv,
                                                               Wv * Cv)
    vdims, vops = [], [xv]
    vspecs = [pl.BlockSpec((Nbv, Tv, Hv, Wv * Cv), lambda n: (n, 0, 0, 0))]
    pool_pads = (1, 1, 0, 1, 1, 1)
    T, Hh, Ww, Cin = Tv, Hv, Wv, Cv
    for p, (w, b) in zip(pool_pads, vconvs):
        Cout = w.shape[4]
        vdims.append((T, Hh, Ww, Cin, Cout, p))
        vops += [_band3d(w, Ww).astype(jnp.bfloat16), jnp.tile(b, Ww)[None, :]]
        vspecs += [pl.BlockSpec(vops[-2].shape, lambda n: (0, 0, 0, 0)),
                   pl.BlockSpec(vops[-1].shape, lambda n: (0, 0))]
        T = T // 2
        Hh = (Hh + 2 * p - 2) // 2 + 1
        Ww = (Ww + 2 * p - 2) // 2 + 1
        Cin = Cout
    vops += [tf_a, fc_w.T, fc_b[None, :]]
    vspecs += [pl.BlockSpec((Nbv,) + tf_a.shape[1:], lambda n: (n, 0, 0)),
               pl.BlockSpec(fc_w.T.shape, lambda n: (0, 0)),
               pl.BlockSpec((1, nclass), lambda n: (0, 0))]
    tf_v, frame3 = pl.pallas_call(
        _make_video_body(Nbv, vdims, C, nclass),
        out_shape=[jax.ShapeDtypeStruct((N, T, C), feature.dtype),
                   jax.ShapeDtypeStruct((N, 1, nclass), feature.dtype)],
        grid_spec=pltpu.PrefetchScalarGridSpec(
            num_scalar_prefetch=0, grid=(N // Nbv,), in_specs=vspecs,
            out_specs=[pl.BlockSpec((Nbv, T, C), lambda n: (n, 0, 0)),
                       pl.BlockSpec((Nbv, 1, nclass), lambda n: (n, 0, 0))]),
        compiler_params=pltpu.CompilerParams(
            dimension_semantics=("parallel",),
            vmem_limit_bytes=_VMEM_LIMIT),
    )(*vops)

    frame = frame3[:, 0, :]
    framewise = jnp.repeat(frame[:, None, :], _UPSAMPLE, axis=1)
    return {"framewise_output": framewise, "clipwise_output": frame,
            "tf_maps_a": tf_a, "tf_maps_v": tf_v}
